# Initial kernel scaffold; baseline (speedup 1.0000x reference)
import jax
import jax.numpy as jnp
from jax.experimental import pallas as pl


def kernel(x, sem_labels, embedding_weight, bbox):
    raise NotImplementedError("write your pallas kernel here")


# speedup vs baseline: 2.7961x; 2.7961x over previous
import functools

import jax
import jax.numpy as jnp
from jax import lax
from jax.experimental import pallas as pl
from jax.experimental.pallas import tpu as pltpu
from jax.experimental.pallas import tpu_sc as plsc

_NUM_WORKERS = 32
_CHUNK = 128


@functools.partial(jax.jit, static_argnums=(3,))
def _sc_embed_concat(x2, labels, table, n_tokens):
    d_x = x2.shape[1]
    d_e = table.shape[1]
    d_out = d_x + d_e
    per_w = n_tokens // _NUM_WORKERS
    steps = per_w // _CHUNK
    mesh = plsc.VectorSubcoreMesh(core_axis_name="c", subcore_axis_name="s")

    @functools.partial(
        pl.kernel,
        mesh=mesh,
        out_type=jax.ShapeDtypeStruct((n_tokens, d_out), jnp.float32),
        scratch_types=[
            pltpu.VMEM((_CHUNK,), jnp.int32),
            pltpu.VMEM((_CHUNK, d_e), jnp.float32),
            pltpu.VMEM((_CHUNK, d_x), jnp.float32),
            pltpu.SemaphoreType.DMA,
        ],
    )
    def k(x_hbm, lab_hbm, tab_hbm, out_hbm, idx_v, emb_v, x_v, sem):
        wid = lax.axis_index("s") * 2 + lax.axis_index("c")
        base_w = wid * per_w

        def body(i, carry):
            base = base_w + i * _CHUNK
            pltpu.sync_copy(lab_hbm.at[pl.ds(base, _CHUNK)], idx_v)
            gather = pltpu.async_copy(tab_hbm.at[idx_v], emb_v, sem)
            pltpu.sync_copy(x_hbm.at[pl.ds(base, _CHUNK), :], x_v)
            pltpu.sync_copy(x_v, out_hbm.at[pl.ds(base, _CHUNK), pl.ds(0, d_x)])
            gather.wait()
            pltpu.sync_copy(emb_v, out_hbm.at[pl.ds(base, _CHUNK), pl.ds(d_x, d_e)])
            return carry

        lax.fori_loop(0, steps, body, 0)

    return k(x2, labels, table)


def kernel(x, sem_labels, embedding_weight, bbox):
    b, t, d_x = x.shape
    n = b * t
    x2 = x.reshape(n, d_x)
    labels = sem_labels.reshape(n).astype(jnp.int32)
    out2 = _sc_embed_concat(x2, labels, embedding_weight, n)
    return out2.reshape(b, t, d_x + embedding_weight.shape[1])
